# baseline (speedup 1.0000x reference)
import functools

import jax
import jax.numpy as jnp
import numpy as np
from jax import lax
from jax.experimental import pallas as pl
from jax.experimental.pallas import tpu as pltpu
from jax.experimental.pallas import tpu_sc as plsc

HIDDEN = 256
NUM_FILTERS = 16
NUM_HEADS = 16
N_NODES = 10000
N_EDGES = 160000
SHIFT = float(np.log(2.0))

NC = 2
NS = 16
DH = HIDDEN // NC
BATCH = 80
CHUNK_A = 72960
CHUNK_B = N_EDGES - CHUNK_A
TB = 256
STRIPE = 640
LAST_STRIPE = N_NODES - 15 * STRIPE


def _transpose_body(in_ref, o_ref):
    o_ref[...] = in_ref[...].T


def _edge_major_chunk(et, lo, n):
    blk0 = lo // TB
    return pl.pallas_call(
        _transpose_body,
        grid=(n // TB,),
        in_specs=[pl.BlockSpec((HIDDEN, TB), lambda i: (0, blk0 + i))],
        out_specs=pl.BlockSpec((TB, HIDDEN), lambda i: (i, 0)),
        out_shape=jax.ShapeDtypeStruct((n, HIDDEN), jnp.float32),
    )(et)


def _sc_segment_sum(e2, idx3, zrows, nblk):
    eps = nblk * BATCH
    mesh = plsc.VectorSubcoreMesh(core_axis_name="c", subcore_axis_name="s")

    @functools.partial(
        pl.kernel,
        mesh=mesh,
        out_type=jax.ShapeDtypeStruct((NC, N_NODES, DH), jnp.float32),
        scratch_types=[
            pltpu.VMEM((nblk, BATCH), jnp.int32),
            pltpu.VMEM((BATCH, DH), jnp.float32),
            pltpu.VMEM((BATCH, DH), jnp.float32),
            pltpu.VMEM_SHARED((N_NODES, DH), jnp.float32),
            pltpu.SemaphoreType.DMA,
            pltpu.SemaphoreType.DMA,
        ],
    )
    def k(e_hbm, idx_hbm, z_hbm, out_hbm, idx_v, e_v0, e_v1, acc, sem0, sem1):
        c = lax.axis_index("c")
        s = lax.axis_index("s")

        @pl.when(s < NS - 1)
        def _():
            pltpu.sync_copy(z_hbm, acc.at[pl.ds(s * STRIPE, STRIPE)])

        @pl.when(s == NS - 1)
        def _():
            pltpu.sync_copy(z_hbm.at[pl.ds(0, LAST_STRIPE)],
                            acc.at[pl.ds(s * STRIPE, LAST_STRIPE)])

        pltpu.sync_copy(idx_hbm.at[s], idx_v)
        plsc.subcore_barrier()

        def gather(j, buf, sem):
            base = s * eps + j * BATCH
            return pltpu.make_async_copy(
                e_hbm.at[pl.ds(base, BATCH), pl.ds(c * DH, DH)], buf, sem)

        def scatter(j, buf):
            pltpu.sync_copy(buf, acc.at[idx_v.at[j]], add=True)

        gather(0, e_v0, sem0).start()
        if nblk % 2:
            @pl.loop(0, nblk - 1, step=2)
            def _(j):
                gather(j + 1, e_v1, sem1).start()
                gather(j, e_v0, sem0).wait()
                scatter(j, e_v0)
                gather(j + 2, e_v0, sem0).start()
                gather(j + 1, e_v1, sem1).wait()
                scatter(j + 1, e_v1)

            gather(nblk - 1, e_v0, sem0).wait()
            scatter(nblk - 1, e_v0)
        else:
            @pl.loop(0, nblk - 2, step=2)
            def _(j):
                gather(j + 1, e_v1, sem1).start()
                gather(j, e_v0, sem0).wait()
                scatter(j, e_v0)
                gather(j + 2, e_v0, sem0).start()
                gather(j + 1, e_v1, sem1).wait()
                scatter(j + 1, e_v1)

            gather(nblk - 1, e_v1, sem1).start()
            gather(nblk - 2, e_v0, sem0).wait()
            scatter(nblk - 2, e_v0)
            gather(nblk - 1, e_v1, sem1).wait()
            scatter(nblk - 1, e_v1)

        plsc.subcore_barrier()

        @pl.when(s < NS - 1)
        def _():
            pltpu.sync_copy(acc.at[pl.ds(s * STRIPE, STRIPE)],
                            out_hbm.at[c, pl.ds(s * STRIPE, STRIPE)])

        @pl.when(s == NS - 1)
        def _():
            pltpu.sync_copy(acc.at[pl.ds(s * STRIPE, LAST_STRIPE)],
                            out_hbm.at[c, pl.ds(s * STRIPE, LAST_STRIPE)])

    return k(e2, idx3, zrows)


def _mlp_body(a0_ref, a1_ref, b0_ref, b1h_ref, v_ref, w1a_ref, w1b_ref,
              bias1_ref, w2_ref, bias2_ref, o_ref):
    dn = (((1,), (1,)), ((), ()))
    h = (
        lax.dot_general(a0_ref[0] + b0_ref[0], w1a_ref[...], dn,
                        preferred_element_type=jnp.float32)
        + lax.dot_general(a1_ref[0] + b1h_ref[0], w1b_ref[...], dn,
                          preferred_element_type=jnp.float32)
        + bias1_ref[...]
    )
    sp = jnp.logaddexp(h, 0.0) - SHIFT
    o_ref[...] = (
        lax.dot_general(sp, w2_ref[...], dn,
                        preferred_element_type=jnp.float32)
        + bias2_ref[...]
        + v_ref[...]
    )


def _mlp(agg_a, agg_b, v, w1a, w1b, b1, w2, b2):
    rows = 1000
    grid = (N_NODES // rows,)
    half = lambda ci: pl.BlockSpec((1, rows, DH), lambda i, ci=ci: (ci, i, 0))
    return pl.pallas_call(
        _mlp_body,
        grid=grid,
        in_specs=[
            half(0), half(1), half(0), half(1),
            pl.BlockSpec((rows, HIDDEN), lambda i: (i, 0)),
            pl.BlockSpec((HIDDEN, DH), lambda i: (0, 0)),
            pl.BlockSpec((HIDDEN, DH), lambda i: (0, 0)),
            pl.BlockSpec((1, HIDDEN), lambda i: (0, 0)),
            pl.BlockSpec((HIDDEN, HIDDEN), lambda i: (0, 0)),
            pl.BlockSpec((1, HIDDEN), lambda i: (0, 0)),
        ],
        out_specs=pl.BlockSpec((rows, HIDDEN), lambda i: (i, 0)),
        out_shape=jax.ShapeDtypeStruct((N_NODES, HIDDEN), jnp.float32),
    )(agg_a, agg_a, agg_b, agg_b, v, w1a, w1b, b1, w2, b2)


def kernel(v, e, edge_index, W1, b1, W2, b2):
    idx = edge_index[1].astype(jnp.int32)
    et = e.reshape(N_EDGES, HIDDEN).T
    zrows = jnp.zeros((STRIPE, DH), jnp.float32)

    nblk_a = CHUNK_A // (NS * BATCH)
    nblk_b = CHUNK_B // (NS * BATCH)
    ea = _edge_major_chunk(et, 0, CHUNK_A)
    eb = _edge_major_chunk(et, CHUNK_A, CHUNK_B)
    agg_a = _sc_segment_sum(
        ea, idx[:CHUNK_A].reshape(NS, nblk_a, BATCH), zrows, nblk_a)
    agg_b = _sc_segment_sum(
        eb, idx[CHUNK_A:].reshape(NS, nblk_b, BATCH), zrows, nblk_b)
    return _mlp(agg_a, agg_b, v, W1[:, :DH], W1[:, DH:],
                b1.reshape(1, HIDDEN), W2, b2.reshape(1, HIDDEN))


# speedup vs baseline: 1.5654x; 1.5654x over previous
import functools

import jax
import jax.numpy as jnp
import numpy as np
from jax import lax
from jax.experimental import pallas as pl
from jax.experimental.pallas import tpu as pltpu
from jax.experimental.pallas import tpu_sc as plsc

HIDDEN = 256
NUM_FILTERS = 16
NUM_HEADS = 16
N_NODES = 10000
N_EDGES = 160000
SHIFT = float(np.log(2.0))

NC = 2
NS = 16
DH = HIDDEN // NC
BATCH = 80
CHUNK_A = 72960
CHUNK_B = N_EDGES - CHUNK_A
TB = 640
STRIPE = 640
LAST_STRIPE = N_NODES - 15 * STRIPE


def _transpose_body(in_ref, o_ref):
    o_ref[...] = in_ref[...].T


def _edge_major_chunk(et, lo, n):
    blk0 = lo // TB
    return pl.pallas_call(
        _transpose_body,
        grid=(n // TB,),
        in_specs=[pl.BlockSpec((HIDDEN, TB), lambda i: (0, blk0 + i))],
        out_specs=pl.BlockSpec((TB, HIDDEN), lambda i: (i, 0)),
        out_shape=jax.ShapeDtypeStruct((n, HIDDEN), jnp.float32),
        compiler_params=pltpu.CompilerParams(
            dimension_semantics=("parallel",)),
    )(et)


def _sc_segment_sum(e2, idx3, zrows, nblk):
    eps = nblk * BATCH
    mesh = plsc.VectorSubcoreMesh(core_axis_name="c", subcore_axis_name="s")

    @functools.partial(
        pl.kernel,
        mesh=mesh,
        out_type=jax.ShapeDtypeStruct((NC, N_NODES, DH), jnp.float32),
        scratch_types=[
            pltpu.VMEM((nblk, BATCH), jnp.int32),
            pltpu.VMEM((BATCH, DH), jnp.float32),
            pltpu.VMEM((BATCH, DH), jnp.float32),
            pltpu.VMEM_SHARED((N_NODES, DH), jnp.float32),
            pltpu.SemaphoreType.DMA,
            pltpu.SemaphoreType.DMA,
        ],
    )
    def k(e_hbm, idx_hbm, z_hbm, out_hbm, idx_v, e_v0, e_v1, acc, sem0, sem1):
        c = lax.axis_index("c")
        s = lax.axis_index("s")

        @pl.when(s < NS - 1)
        def _():
            pltpu.sync_copy(z_hbm, acc.at[pl.ds(s * STRIPE, STRIPE)])

        @pl.when(s == NS - 1)
        def _():
            pltpu.sync_copy(z_hbm.at[pl.ds(0, LAST_STRIPE)],
                            acc.at[pl.ds(s * STRIPE, LAST_STRIPE)])

        pltpu.sync_copy(idx_hbm.at[s], idx_v)
        plsc.subcore_barrier()

        def gather(j, buf, sem):
            base = s * eps + j * BATCH
            return pltpu.make_async_copy(
                e_hbm.at[pl.ds(base, BATCH), pl.ds(c * DH, DH)], buf, sem)

        def scatter(j, buf):
            pltpu.sync_copy(buf, acc.at[idx_v.at[j]], add=True)

        gather(0, e_v0, sem0).start()
        if nblk % 2:
            @pl.loop(0, nblk - 1, step=2)
            def _(j):
                gather(j + 1, e_v1, sem1).start()
                gather(j, e_v0, sem0).wait()
                scatter(j, e_v0)
                gather(j + 2, e_v0, sem0).start()
                gather(j + 1, e_v1, sem1).wait()
                scatter(j + 1, e_v1)

            gather(nblk - 1, e_v0, sem0).wait()
            scatter(nblk - 1, e_v0)
        else:
            @pl.loop(0, nblk - 2, step=2)
            def _(j):
                gather(j + 1, e_v1, sem1).start()
                gather(j, e_v0, sem0).wait()
                scatter(j, e_v0)
                gather(j + 2, e_v0, sem0).start()
                gather(j + 1, e_v1, sem1).wait()
                scatter(j + 1, e_v1)

            gather(nblk - 1, e_v1, sem1).start()
            gather(nblk - 2, e_v0, sem0).wait()
            scatter(nblk - 2, e_v0)
            gather(nblk - 1, e_v1, sem1).wait()
            scatter(nblk - 1, e_v1)

        plsc.subcore_barrier()

        @pl.when(s < NS - 1)
        def _():
            pltpu.sync_copy(acc.at[pl.ds(s * STRIPE, STRIPE)],
                            out_hbm.at[c, pl.ds(s * STRIPE, STRIPE)])

        @pl.when(s == NS - 1)
        def _():
            pltpu.sync_copy(acc.at[pl.ds(s * STRIPE, LAST_STRIPE)],
                            out_hbm.at[c, pl.ds(s * STRIPE, LAST_STRIPE)])

    return k(e2, idx3, zrows)


def _mlp_body(a0_ref, a1_ref, b0_ref, b1h_ref, v_ref, w1a_ref, w1b_ref,
              bias1_ref, w2_ref, bias2_ref, o_ref):
    dn = (((1,), (1,)), ((), ()))
    h = (
        lax.dot_general(a0_ref[0] + b0_ref[0], w1a_ref[...], dn,
                        preferred_element_type=jnp.float32)
        + lax.dot_general(a1_ref[0] + b1h_ref[0], w1b_ref[...], dn,
                          preferred_element_type=jnp.float32)
        + bias1_ref[...]
    )
    sp = jnp.logaddexp(h, 0.0) - SHIFT
    o_ref[...] = (
        lax.dot_general(sp, w2_ref[...], dn,
                        preferred_element_type=jnp.float32)
        + bias2_ref[...]
        + v_ref[...]
    )


def _mlp(agg_a, agg_b, v, w1a, w1b, b1, w2, b2):
    rows = 1000
    grid = (N_NODES // rows,)
    half = lambda ci: pl.BlockSpec((1, rows, DH), lambda i, ci=ci: (ci, i, 0))
    return pl.pallas_call(
        _mlp_body,
        grid=grid,
        in_specs=[
            half(0), half(1), half(0), half(1),
            pl.BlockSpec((rows, HIDDEN), lambda i: (i, 0)),
            pl.BlockSpec((HIDDEN, DH), lambda i: (0, 0)),
            pl.BlockSpec((HIDDEN, DH), lambda i: (0, 0)),
            pl.BlockSpec((1, HIDDEN), lambda i: (0, 0)),
            pl.BlockSpec((HIDDEN, HIDDEN), lambda i: (0, 0)),
            pl.BlockSpec((1, HIDDEN), lambda i: (0, 0)),
        ],
        out_specs=pl.BlockSpec((rows, HIDDEN), lambda i: (i, 0)),
        out_shape=jax.ShapeDtypeStruct((N_NODES, HIDDEN), jnp.float32),
        compiler_params=pltpu.CompilerParams(
            dimension_semantics=("parallel",)),
    )(agg_a, agg_a, agg_b, agg_b, v, w1a, w1b, b1, w2, b2)


def kernel(v, e, edge_index, W1, b1, W2, b2):
    idx = edge_index[1].astype(jnp.int32)
    et = e.reshape(N_EDGES, HIDDEN).T
    zrows = jnp.zeros((STRIPE, DH), jnp.float32)

    nblk_a = CHUNK_A // (NS * BATCH)
    nblk_b = CHUNK_B // (NS * BATCH)
    ea = _edge_major_chunk(et, 0, CHUNK_A)
    eb = _edge_major_chunk(et, CHUNK_A, CHUNK_B)
    agg_a = _sc_segment_sum(
        ea, idx[:CHUNK_A].reshape(NS, nblk_a, BATCH), zrows, nblk_a)
    agg_b = _sc_segment_sum(
        eb, idx[CHUNK_A:].reshape(NS, nblk_b, BATCH), zrows, nblk_b)
    return _mlp(agg_a, agg_b, v, W1[:, :DH], W1[:, DH:],
                b1.reshape(1, HIDDEN), W2, b2.reshape(1, HIDDEN))
